# Initial kernel scaffold; baseline (speedup 1.0000x reference)
import jax
import jax.numpy as jnp
from jax.experimental import pallas as pl


def kernel(x, edge_index, batch, enc_W1, enc_b1, enc_W2, enc_b2, gW0, gb0, gW1, gb1, dec_W1, dec_b1, dec_W2, dec_b2):
    raise NotImplementedError("write your pallas kernel here")


# speedup vs baseline: 10.2143x; 10.2143x over previous
import dataclasses
import functools

import jax
import jax.numpy as jnp
from jax import lax
from jax.experimental import pallas as pl
from jax.experimental.pallas import tpu as pltpu
from jax.experimental.pallas import tpu_sc as plsc

N = 10000
E = 320000
D = 128
H = 128
OUT = 128
G = 64

NC = 2
NS = 16
NW = NC * NS
LANES = 16

C = 128
CPT = -(-(E // NW) // C)
EPT = CPT * C
EP = EPT * NW
NP = 10240
RPT = NP // NS
NL = 10016

BR = 1000
NBLK = N // BR

_mesh = plsc.VectorSubcoreMesh(core_axis_name="c", subcore_axis_name="s")

_sc_params = pltpu.CompilerParams()
if "needs_layout_passes" in pltpu.CompilerParams.__dataclass_fields__:
    _sc_params = dataclasses.replace(_sc_params, needs_layout_passes=False)

F32 = jnp.float32
HIGH = lax.Precision.HIGHEST


@jax.jit
def _degree_partials(dst2d):
    @functools.partial(
        pl.kernel,
        out_type=jax.ShapeDtypeStruct((NW, NL), F32),
        mesh=_mesh,
        compiler_params=_sc_params,
        scratch_types=[
            pltpu.VMEM((C,), jnp.int32),
            pltpu.VMEM((NL,), F32),
        ],
    )
    def deg_kernel(dst_hbm, out_hbm, idx_v, hist_v):
        cid = lax.axis_index("c")
        sid = lax.axis_index("s")
        wid = cid * NS + sid
        zeros16 = jnp.zeros((LANES,), F32)
        ones16 = jnp.ones((LANES,), F32)

        @pl.loop(0, NL, step=LANES)
        def _(i):
            hist_v[pl.ds(i, LANES)] = zeros16

        @pl.loop(0, CPT)
        def _(c):
            pltpu.sync_copy(dst_hbm.at[wid * CPT + c], idx_v)
            for j in range(C // LANES):
                iv = idx_v[pl.ds(j * LANES, LANES)]
                plsc.addupdate_scatter(hist_v, [iv], ones16)

        pltpu.sync_copy(hist_v, out_hbm.at[wid])

    return deg_kernel(dst2d)


@jax.jit
def _edge_scatter(y, src2d, dst2d):
    @functools.partial(
        pl.kernel,
        out_type=jax.ShapeDtypeStruct((NC, NP, D), F32),
        mesh=_mesh,
        compiler_params=_sc_params,
        scratch_types=[
            pltpu.VMEM((C,), jnp.int32),
            pltpu.VMEM((C,), jnp.int32),
            pltpu.VMEM((C, D), F32),
            pltpu.VMEM((64, D), F32),
            pltpu.VMEM_SHARED((NP, D), F32),
            pltpu.SemaphoreType.DMA,
        ],
    )
    def scat_kernel(y_hbm, src_hbm, dst_hbm, out_hbm,
                    srcv, dstv, rows, zbuf, acc, sem):
        cid = lax.axis_index("c")
        sid = lax.axis_index("s")
        wid = cid * NS + sid
        zeros16 = jnp.zeros((LANES,), F32)

        @pl.loop(0, 64)
        def _(r):
            for j in range(D // LANES):
                zbuf[r, pl.ds(j * LANES, LANES)] = zeros16

        @pl.loop(0, RPT // 64)
        def _(k):
            pltpu.sync_copy(zbuf, acc.at[pl.ds(sid * RPT + k * 64, 64)])

        plsc.subcore_barrier()

        @pl.loop(0, CPT)
        def _(c):
            base = wid * CPT + c
            pltpu.sync_copy(src_hbm.at[base], srcv)
            pltpu.sync_copy(dst_hbm.at[base], dstv)
            pltpu.async_copy(y_hbm.at[srcv], rows, sem).wait()
            pltpu.sync_copy(rows, acc.at[dstv], add=True)

        plsc.subcore_barrier()
        pltpu.sync_copy(acc.at[pl.ds(sid * RPT, RPT)],
                        out_hbm.at[cid, pl.ds(sid * RPT, RPT)])

    return scat_kernel(y, src2d, dst2d)


def _dinv_of(hist_blk):
    deg = jnp.sum(hist_blk, axis=1) + 1.0
    return lax.rsqrt(deg).reshape(-1, 1)


def _prep_body(x_ref, hist_ref, w1_ref, b1_ref, w2_ref, b2_ref, gw_ref, y_ref):
    dinv = _dinv_of(hist_ref[...])
    h = jnp.maximum(
        jnp.dot(x_ref[...], w1_ref[...], preferred_element_type=F32,
                precision=HIGH) + b1_ref[...], 0.0)
    h = jnp.dot(h, w2_ref[...], preferred_element_type=F32,
                precision=HIGH) + b2_ref[...]
    xw = jnp.dot(h, gw_ref[...], preferred_element_type=F32, precision=HIGH)
    y_ref[...] = xw * dinv


@jax.jit
def _prep(x, hist, w1, b1, w2, b2, gw):
    return pl.pallas_call(
        _prep_body,
        grid=(NBLK,),
        in_specs=[
            pl.BlockSpec((BR, D), lambda i: (i, 0)),
            pl.BlockSpec((BR, NW), lambda i: (i, 0)),
            pl.BlockSpec((D, H), lambda i: (0, 0)),
            pl.BlockSpec((1, H), lambda i: (0, 0)),
            pl.BlockSpec((H, H), lambda i: (0, 0)),
            pl.BlockSpec((1, H), lambda i: (0, 0)),
            pl.BlockSpec((H, H), lambda i: (0, 0)),
        ],
        out_specs=pl.BlockSpec((BR, H), lambda i: (i, 0)),
        out_shape=jax.ShapeDtypeStruct((N, H), F32),
    )(x, hist, w1, b1, w2, b2, gw)


def _mid_body(acc_ref, y_ref, hist_ref, b_ref, gw_ref, y2_ref):
    dinv = _dinv_of(hist_ref[...])
    s = acc_ref[0] + acc_ref[1] + y_ref[...]
    h = jnp.maximum(s * dinv + b_ref[...], 0.0)
    y2_ref[...] = jnp.dot(h, gw_ref[...], preferred_element_type=F32,
                          precision=HIGH) * dinv


@jax.jit
def _mid(acc, y, hist, b, gw):
    return pl.pallas_call(
        _mid_body,
        grid=(NBLK,),
        in_specs=[
            pl.BlockSpec((NC, BR, H), lambda i: (0, i, 0)),
            pl.BlockSpec((BR, H), lambda i: (i, 0)),
            pl.BlockSpec((BR, NW), lambda i: (i, 0)),
            pl.BlockSpec((1, H), lambda i: (0, 0)),
            pl.BlockSpec((H, H), lambda i: (0, 0)),
        ],
        out_specs=pl.BlockSpec((BR, H), lambda i: (i, 0)),
        out_shape=jax.ShapeDtypeStruct((N, H), F32),
    )(acc, y, hist, b, gw)


def _final_body(acc_ref, y_ref, hist_ref, batch_ref, b_ref,
                dw1_ref, db1_ref, dw2_ref, db2_ref, out_ref, pool_scr):
    i = pl.program_id(0)
    dinv = _dinv_of(hist_ref[...])
    s = acc_ref[0] + acc_ref[1] + y_ref[...]
    h = jnp.maximum(s * dinv + b_ref[...], 0.0)
    b = batch_ref[0, 0]
    oh = (b[:, None] == lax.broadcasted_iota(jnp.int32, (BR, G), 1)).astype(F32)
    part = lax.dot_general(oh, h, (((0,), (0,)), ((), ())),
                           preferred_element_type=F32, precision=HIGH)

    @pl.when(i == 0)
    def _():
        pool_scr[...] = part

    @pl.when(i > 0)
    def _():
        pool_scr[...] += part

    @pl.when(i == NBLK - 1)
    def _():
        pooled = pool_scr[...]
        d = jnp.maximum(
            jnp.dot(pooled, dw1_ref[...], preferred_element_type=F32,
                    precision=HIGH) + db1_ref[...], 0.0)
        out_ref[...] = jnp.dot(d, dw2_ref[...], preferred_element_type=F32,
                               precision=HIGH) + db2_ref[...]


@jax.jit
def _final(acc, y, hist, batch3, b, dw1, db1, dw2, db2):
    return pl.pallas_call(
        _final_body,
        grid=(NBLK,),
        in_specs=[
            pl.BlockSpec((NC, BR, H), lambda i: (0, i, 0)),
            pl.BlockSpec((BR, H), lambda i: (i, 0)),
            pl.BlockSpec((BR, NW), lambda i: (i, 0)),
            pl.BlockSpec((1, 1, BR), lambda i: (i, 0, 0)),
            pl.BlockSpec((1, H), lambda i: (0, 0)),
            pl.BlockSpec((H, H), lambda i: (0, 0)),
            pl.BlockSpec((1, H), lambda i: (0, 0)),
            pl.BlockSpec((H, OUT), lambda i: (0, 0)),
            pl.BlockSpec((1, OUT), lambda i: (0, 0)),
        ],
        out_specs=pl.BlockSpec((G, OUT), lambda i: (0, 0)),
        out_shape=jax.ShapeDtypeStruct((G, OUT), F32),
        scratch_shapes=[pltpu.VMEM((G, H), F32)],
    )(acc, y, hist, batch3, b, dw1, db1, dw2, db2)


@jax.jit
def kernel(x, edge_index, batch, enc_W1, enc_b1, enc_W2, enc_b2,
           gW0, gb0, gW1, gb1, dec_W1, dec_b1, dec_W2, dec_b2):
    src = edge_index[0]
    dst = edge_index[1]
    pad = EP - E
    src2d = jnp.concatenate(
        [src, jnp.zeros((pad,), jnp.int32)]).reshape(NW * CPT, C)
    dst2d = jnp.concatenate(
        [dst, jnp.full((pad,), N, jnp.int32)]).reshape(NW * CPT, C)
    batch3 = batch.reshape(NBLK, 1, BR)

    hist = _degree_partials(dst2d).T

    y1 = _prep(x, hist, enc_W1, enc_b1.reshape(1, H), enc_W2,
               enc_b2.reshape(1, H), gW0)
    acc1 = _edge_scatter(y1, src2d, dst2d)
    y2 = _mid(acc1, y1, hist, gb0.reshape(1, H), gW1)
    acc2 = _edge_scatter(y2, src2d, dst2d)
    return _final(acc2, y2, hist, batch3, gb1.reshape(1, H),
                  dec_W1, dec_b1.reshape(1, H), dec_W2, dec_b2.reshape(1, OUT))
